# baseline (speedup 1.0000x reference)
import functools

import jax
import jax.numpy as jnp
from jax.experimental import pallas as pl
from jax.experimental.pallas import tpu as pltpu

T, B, C, H, W = 4, 16, 256, 14, 14
E, K = 8, 2
HID, OUT = 256, 256
HW = H * W
TB = T * B
_C1 = 1.0 / (1.0 + 1e-5) ** 0.5


def _router_body(x_ref, wr_ref, shift_ref, out_ref):
    v = jnp.zeros((C, HW), jnp.float32)
    ms = []
    for t in range(T):
        v = (v + x_ref[t, 0]) * 0.5
        s = (v >= 1.0).astype(jnp.float32)
        v = v * (1.0 - s)
        ms.append(jnp.sum(s, axis=-1))
    m = jnp.stack(ms, axis=0) * (1.0 / HW)
    out_ref[0] = (
        jnp.dot(m, wr_ref[...], preferred_element_type=jnp.float32) + shift_ref[...]
    )


CHUNK = 8


def _expert_body(idx_ref, tau_ref, wk_ref, x_ref, w1_ref, w2_ref, d1_ref, d2_ref,
                 out_ref):
    n = pl.program_id(0)
    for j in range(CHUNK):
        t = n * CHUNK + j
        x = x_ref[j]
        acc = None
        for k in range(K):
            e = idx_ref[t * K + k]
            tau = tau_ref[t * K + k]
            wgt = wk_ref[t * K + k]
            s1 = (x >= tau).astype(jnp.float32)
            h = (jnp.dot(w1_ref[e], s1, preferred_element_type=jnp.float32)
                 + d1_ref[e, 0][:, None])
            x2 = x + h
            s2 = (x2 >= tau).astype(jnp.float32)
            o = (jnp.dot(w2_ref[e], s2, preferred_element_type=jnp.float32)
                 + d2_ref[e, 0][:, None])
            res = (o + x2) * wgt
            acc = res if k == 0 else acc + res
        out_ref[j] = acc


def kernel(x, Wr, br, gr, betar, W1, b1, g1, bt1, W2, b2, g2, bt2):
    f32 = jnp.float32
    x4 = x.reshape(T, B, C, HW)

    wr_s = Wr.T * (gr * _C1)[None, :]
    shift = (br * gr * _C1 + betar)[None, :]
    logits_bt = pl.pallas_call(
        _router_body,
        grid=(B,),
        in_specs=[
            pl.BlockSpec((T, 1, C, HW), lambda b: (0, b, 0, 0)),
            pl.BlockSpec((C, E), lambda b: (0, 0)),
            pl.BlockSpec((1, E), lambda b: (0, 0)),
        ],
        out_specs=pl.BlockSpec((1, T, E), lambda b: (b, 0, 0)),
        out_shape=jax.ShapeDtypeStruct((B, T, E), f32),
    )(x4, wr_s, shift)
    logits = logits_bt.transpose(1, 0, 2).reshape(TB, E)

    probs = jax.nn.softmax(logits, axis=-1)
    wk, idx = jax.lax.top_k(probs, K)
    wk = wk / jnp.sum(wk, axis=-1, keepdims=True)

    taus = jnp.linspace(1.5, 4.0, E).astype(f32)
    idx_p = idx.reshape(-1).astype(jnp.int32)
    tau_p = taus[idx_p]
    wk_p = wk.reshape(-1).astype(f32)

    w1g = W1 * (g1 * _C1)[:, :, None]
    w2g = W2 * (g2 * _C1)[:, :, None]
    d1 = (b1 * g1 * _C1 + bt1).reshape(E, 1, HID)
    d2 = (b2 * g2 * _C1 + bt2).reshape(E, 1, OUT)
    xt = x4.reshape(TB, C, HW)

    out = pl.pallas_call(
        _expert_body,
        grid_spec=pltpu.PrefetchScalarGridSpec(
            num_scalar_prefetch=3,
            grid=(TB // CHUNK,),
            in_specs=[
                pl.BlockSpec((CHUNK, C, HW), lambda n, i, ta, wv: (n, 0, 0)),
                pl.BlockSpec((E, HID, C), lambda n, i, ta, wv: (0, 0, 0)),
                pl.BlockSpec((E, OUT, HID), lambda n, i, ta, wv: (0, 0, 0)),
                pl.BlockSpec((E, 1, HID), lambda n, i, ta, wv: (0, 0, 0)),
                pl.BlockSpec((E, 1, OUT), lambda n, i, ta, wv: (0, 0, 0)),
            ],
            out_specs=pl.BlockSpec((CHUNK, OUT, HW), lambda n, i, ta, wv: (n, 0, 0)),
        ),
        out_shape=jax.ShapeDtypeStruct((TB, OUT, HW), f32),
    )(idx_p, tau_p, wk_p, xt, w1g, w2g, d1, d2)

    return out.reshape(T, B, OUT, H, W)


# speedup vs baseline: 1.0060x; 1.0060x over previous
import functools

import jax
import jax.numpy as jnp
from jax.experimental import pallas as pl
from jax.experimental.pallas import tpu as pltpu

T, B, C, H, W = 4, 16, 256, 14, 14
E, K = 8, 2
HID, OUT = 256, 256
HW = H * W
TB = T * B
_C1 = 1.0 / (1.0 + 1e-5) ** 0.5


def _router_body(x_ref, wr_ref, shift_ref, tau_tab_ref, idx_ref, tau_ref, wk_ref):
    v = jnp.zeros((C, HW), jnp.float32)
    ms = []
    for t in range(T):
        v = (v + x_ref[t, 0]) * 0.5
        s = (v >= 1.0).astype(jnp.float32)
        v = v * (1.0 - s)
        ms.append(jnp.sum(s, axis=-1))
    m = jnp.stack(ms, axis=0) * (1.0 / HW)
    logits = (
        jnp.dot(m, wr_ref[...], preferred_element_type=jnp.float32) + shift_ref[...]
    )
    ie = jax.lax.broadcasted_iota(jnp.int32, (T, E), 1)
    l0 = jnp.max(logits, axis=1, keepdims=True)
    a0 = jnp.min(jnp.where(logits == l0, ie, E), axis=1, keepdims=True)
    masked = jnp.where(ie == a0, -jnp.inf, logits)
    l1 = jnp.max(masked, axis=1, keepdims=True)
    a1 = jnp.min(jnp.where(masked == l1, ie, E), axis=1, keepdims=True)
    idx_ref[0] = jnp.concatenate([a0, a1], axis=1)
    wk_ref[0] = jnp.concatenate(
        [jax.nn.sigmoid(l0 - l1), jax.nn.sigmoid(l1 - l0)], axis=1)
    tau_tab = tau_tab_ref[...]
    tau0 = jnp.sum(jnp.where(ie == a0, tau_tab, 0.0), axis=1, keepdims=True)
    tau1 = jnp.sum(jnp.where(ie == a1, tau_tab, 0.0), axis=1, keepdims=True)
    tau_ref[0] = jnp.concatenate([tau0, tau1], axis=1)


CHUNK = 8


def _expert_body(idx_ref, tau_ref, wk_ref, x_ref, w1_ref, w2_ref, d1_ref, d2_ref,
                 out_ref):
    n = pl.program_id(0)
    for j in range(CHUNK):
        t = n * CHUNK + j
        x = x_ref[j]
        acc = None
        for k in range(K):
            e = idx_ref[t * K + k]
            tau = tau_ref[t * K + k]
            wgt = wk_ref[t * K + k]
            s1 = (x >= tau).astype(jnp.float32)
            h = (jnp.dot(w1_ref[e], s1, preferred_element_type=jnp.float32)
                 + d1_ref[e, 0][:, None])
            x2 = x + h
            s2 = (x2 >= tau).astype(jnp.float32)
            o = (jnp.dot(w2_ref[e], s2, preferred_element_type=jnp.float32)
                 + d2_ref[e, 0][:, None])
            res = (o + x2) * wgt
            acc = res if k == 0 else acc + res
        out_ref[j] = acc


def kernel(x, Wr, br, gr, betar, W1, b1, g1, bt1, W2, b2, g2, bt2):
    f32 = jnp.float32
    x4 = x.reshape(T, B, C, HW)

    wr_s = Wr.T * (gr * _C1)[None, :]
    shift = (br * gr * _C1 + betar)[None, :]
    taus = jnp.linspace(1.5, 4.0, E).astype(f32).reshape(1, E)
    idx_bt, tau_bt, wk_bt = pl.pallas_call(
        _router_body,
        grid=(B,),
        in_specs=[
            pl.BlockSpec((T, 1, C, HW), lambda b: (0, b, 0, 0)),
            pl.BlockSpec((C, E), lambda b: (0, 0)),
            pl.BlockSpec((1, E), lambda b: (0, 0)),
            pl.BlockSpec((1, E), lambda b: (0, 0)),
        ],
        out_specs=[
            pl.BlockSpec((1, T, K), lambda b: (b, 0, 0)),
            pl.BlockSpec((1, T, K), lambda b: (b, 0, 0)),
            pl.BlockSpec((1, T, K), lambda b: (b, 0, 0)),
        ],
        out_shape=[
            jax.ShapeDtypeStruct((B, T, K), jnp.int32),
            jax.ShapeDtypeStruct((B, T, K), f32),
            jax.ShapeDtypeStruct((B, T, K), f32),
        ],
    )(x4, wr_s, shift, taus)
    idx_p = idx_bt.transpose(1, 0, 2).reshape(-1)
    tau_p = tau_bt.transpose(1, 0, 2).reshape(-1)
    wk_p = wk_bt.transpose(1, 0, 2).reshape(-1)

    w1g = W1 * (g1 * _C1)[:, :, None]
    w2g = W2 * (g2 * _C1)[:, :, None]
    d1 = (b1 * g1 * _C1 + bt1).reshape(E, 1, HID)
    d2 = (b2 * g2 * _C1 + bt2).reshape(E, 1, OUT)
    xt = x4.reshape(TB, C, HW)

    out = pl.pallas_call(
        _expert_body,
        grid_spec=pltpu.PrefetchScalarGridSpec(
            num_scalar_prefetch=3,
            grid=(TB // CHUNK,),
            in_specs=[
                pl.BlockSpec((CHUNK, C, HW), lambda n, i, ta, wv: (n, 0, 0)),
                pl.BlockSpec((E, HID, C), lambda n, i, ta, wv: (0, 0, 0)),
                pl.BlockSpec((E, OUT, HID), lambda n, i, ta, wv: (0, 0, 0)),
                pl.BlockSpec((E, 1, HID), lambda n, i, ta, wv: (0, 0, 0)),
                pl.BlockSpec((E, 1, OUT), lambda n, i, ta, wv: (0, 0, 0)),
            ],
            out_specs=pl.BlockSpec((CHUNK, OUT, HW), lambda n, i, ta, wv: (n, 0, 0)),
        ),
        out_shape=jax.ShapeDtypeStruct((TB, OUT, HW), f32),
    )(idx_p, tau_p, wk_p, xt, w1g, w2g, d1, d2)

    return out.reshape(T, B, OUT, H, W)


# speedup vs baseline: 1.6281x; 1.6184x over previous
import jax
import jax.numpy as jnp
from jax.experimental import pallas as pl

T, B, C, H, W = 4, 16, 256, 14, 14
E, K = 8, 2
HID, OUT = 256, 256
HW = H * W
TB = T * B
_C1 = 1.0 / (1.0 + 1e-5) ** 0.5


def _fused_body(x_ref, wr_ref, shift_ref, tau_tab_ref, w1_ref, w2_ref,
                d1_ref, d2_ref, out_ref):
    v = jnp.zeros((C, HW), jnp.float32)
    ms = []
    for tt in range(T):
        v = (v + x_ref[tt, 0]) * 0.5
        s = (v >= 1.0).astype(jnp.float32)
        v = v * (1.0 - s)
        ms.append(jnp.sum(s, axis=-1))
    m = jnp.stack(ms, axis=0) * (1.0 / HW)
    logits = (
        jnp.dot(m, wr_ref[...], preferred_element_type=jnp.float32) + shift_ref[...]
    )

    ie = jax.lax.broadcasted_iota(jnp.int32, (T, E), 1)
    l0 = jnp.max(logits, axis=1, keepdims=True)
    a0 = jnp.min(jnp.where(logits == l0, ie, E), axis=1, keepdims=True)
    masked = jnp.where(ie == a0, -jnp.inf, logits)
    l1 = jnp.max(masked, axis=1, keepdims=True)
    a1 = jnp.min(jnp.where(masked == l1, ie, E), axis=1, keepdims=True)
    w0 = jax.nn.sigmoid(l0 - l1)
    w1 = jax.nn.sigmoid(l1 - l0)
    tau_tab = tau_tab_ref[...]
    tau0 = jnp.sum(jnp.where(ie == a0, tau_tab, 0.0), axis=1, keepdims=True)
    tau1 = jnp.sum(jnp.where(ie == a1, tau_tab, 0.0), axis=1, keepdims=True)

    for tt in range(T):
        x = x_ref[tt, 0]
        acc = None
        for k in range(K):
            a_v, tau_v, wk_v = ((a0, tau0, w0), (a1, tau1, w1))[k]
            e = a_v[tt, 0]
            tau = tau_v[tt:tt + 1, 0:1]
            wgt = wk_v[tt:tt + 1, 0:1]
            s1 = (x >= tau).astype(jnp.float32)
            h = (jnp.dot(w1_ref[e], s1, preferred_element_type=jnp.float32)
                 + d1_ref[e, 0][:, None])
            x2 = x + h
            s2 = (x2 >= tau).astype(jnp.float32)
            o = (jnp.dot(w2_ref[e], s2, preferred_element_type=jnp.float32)
                 + d2_ref[e, 0][:, None])
            res = (o + x2) * wgt
            acc = res if k == 0 else acc + res
        out_ref[tt, 0] = acc


def kernel(x, Wr, br, gr, betar, W1, b1, g1, bt1, W2, b2, g2, bt2):
    f32 = jnp.float32
    x4 = x.reshape(T, B, C, HW)

    wr_s = Wr.T * (gr * _C1)[None, :]
    shift = (br * gr * _C1 + betar)[None, :]
    taus = jnp.linspace(1.5, 4.0, E).astype(f32).reshape(1, E)
    w1g = W1 * (g1 * _C1)[:, :, None]
    w2g = W2 * (g2 * _C1)[:, :, None]
    d1 = (b1 * g1 * _C1 + bt1).reshape(E, 1, HID)
    d2 = (b2 * g2 * _C1 + bt2).reshape(E, 1, OUT)

    out = pl.pallas_call(
        _fused_body,
        grid=(B,),
        in_specs=[
            pl.BlockSpec((T, 1, C, HW), lambda b: (0, b, 0, 0)),
            pl.BlockSpec((C, E), lambda b: (0, 0)),
            pl.BlockSpec((1, E), lambda b: (0, 0)),
            pl.BlockSpec((1, E), lambda b: (0, 0)),
            pl.BlockSpec((E, HID, C), lambda b: (0, 0, 0)),
            pl.BlockSpec((E, OUT, HID), lambda b: (0, 0, 0)),
            pl.BlockSpec((E, 1, HID), lambda b: (0, 0, 0)),
            pl.BlockSpec((E, 1, OUT), lambda b: (0, 0, 0)),
        ],
        out_specs=pl.BlockSpec((T, 1, OUT, HW), lambda b: (0, b, 0, 0)),
        out_shape=jax.ShapeDtypeStruct((T, B, OUT, HW), f32),
    )(x4, wr_s, shift, taus, w1g, w2g, d1, d2)

    return out.reshape(T, B, OUT, H, W)
